# baseline (speedup 1.0000x reference)
import jax
import jax.numpy as jnp
from jax.experimental import pallas as pl

NA = 3
NO = 85


def _detect_kernel(x0_ref, x1_ref, x2_ref,
                   wt0_ref, wt1_ref, wt2_ref, b_ref,
                   out0_ref, out1_ref, out2_ref):
    dn = (((0,), (0,)), ((), ()))
    for x_ref, wt_ref, lvl, out_ref in (
            (x0_ref, wt0_ref, 0, out0_ref),
            (x1_ref, wt1_ref, 1, out1_ref),
            (x2_ref, wt2_ref, 2, out2_ref)):
        c, ny, nx = x_ref.shape[1:]
        xv = x_ref[0].astype(jnp.bfloat16).reshape(c, ny * nx)
        for a in range(NA):
            z = jax.lax.dot_general(xv, wt_ref[a], dn,
                                    preferred_element_type=jnp.float32)
            out_ref[0, a] = (z + b_ref[lvl, a]).reshape(ny, nx, NO)


def _pack_w(W, b):
    c = W.shape[1]
    wt = W.reshape(NA, NO, c).transpose(0, 2, 1).astype(jnp.bfloat16)
    br = b.reshape(NA, NO)
    return wt, br


@jax.jit
def _detect(x0, x1, x2, W0, b0, W1, b1, W2, b2):
    bs = x0.shape[0]
    shapes = [x.shape for x in (x0, x1, x2)]
    packed = [_pack_w(W, b) for W, b in ((W0, b0), (W1, b1), (W2, b2))]
    wts = [p[0] for p in packed]
    brs = jnp.stack([p[1] for p in packed])

    def x_spec(c, ny, nx):
        return pl.BlockSpec((1, c, ny // 2, nx), lambda i, j: (i, 0, j, 0))

    def w_spec(c):
        return pl.BlockSpec((NA, c, NO), lambda i, j: (0, 0, 0))

    def o_spec(ny, nx):
        return pl.BlockSpec((1, NA, ny // 2, nx, NO),
                            lambda i, j: (i, 0, j, 0, 0))

    return pl.pallas_call(
        _detect_kernel,
        grid=(bs, 2),
        in_specs=(
            [x_spec(s[1], s[2], s[3]) for s in shapes]
            + [w_spec(s[1]) for s in shapes]
            + [pl.BlockSpec((3, NA, NO), lambda i, j: (0, 0, 0))]
        ),
        out_specs=[o_spec(s[2], s[3]) for s in shapes],
        out_shape=[
            jax.ShapeDtypeStruct((bs, NA, s[2], s[3], NO), jnp.float32)
            for s in shapes],
    )(x0, x1, x2, *wts, brs)


def kernel(x0, x1, x2, W0, b0, W1, b1, W2, b2):
    return tuple(_detect(x0, x1, x2, W0, b0, W1, b1, W2, b2))


# speedup vs baseline: 1.7619x; 1.7619x over previous
import jax
import jax.numpy as jnp
from jax.experimental import pallas as pl

NA = 3
NO = 85


def _detect_kernel(x0_ref, x1_ref, x2_ref,
                   wt0_ref, wt1_ref, wt2_ref, b_ref,
                   out0_ref, out1_ref, out2_ref):
    dn = (((0,), (0,)), ((), ()))
    c, ny, nx = x0_ref.shape[1:]
    xv = x0_ref[0].astype(jnp.bfloat16).reshape(c, ny * nx)
    for a in range(NA):
        z = jax.lax.dot_general(xv, wt0_ref[a], dn,
                                preferred_element_type=jnp.float32)
        out0_ref[0, a] = (z + b_ref[0, a]).reshape(ny, nx, NO)
    for x_ref, wt_ref, lvl, out_ref in (
            (x1_ref, wt1_ref, 1, out1_ref),
            (x2_ref, wt2_ref, 2, out2_ref)):
        xv = x_ref[0].astype(jnp.bfloat16)
        for a in range(NA):
            z = jax.lax.dot_general(xv, wt_ref[a], dn,
                                    preferred_element_type=jnp.float32)
            out_ref[0, a] = z + b_ref[lvl, a]


def _pack_w(W, b):
    c = W.shape[1]
    wt = W.reshape(NA, NO, c).transpose(0, 2, 1).astype(jnp.bfloat16)
    br = b.reshape(NA, NO)
    return wt, br


@jax.jit
def _detect(x0, x1, x2, W0, b0, W1, b1, W2, b2):
    bs = x0.shape[0]
    shapes = [x.shape for x in (x0, x1, x2)]
    xr = [x.reshape(x.shape[0], x.shape[1], -1) for x in (x1, x2)]
    packed = [_pack_w(W, b) for W, b in ((W0, b0), (W1, b1), (W2, b2))]
    wts = [p[0] for p in packed]
    brs = jnp.stack([p[1] for p in packed])

    def w_spec(c):
        return pl.BlockSpec((NA, c, NO), lambda i: (0, 0, 0))

    s0, s1, s2 = shapes
    outs = pl.pallas_call(
        _detect_kernel,
        grid=(bs,),
        in_specs=(
            [pl.BlockSpec((1, s0[1], s0[2], s0[3]), lambda i: (i, 0, 0, 0)),
             pl.BlockSpec((1, s1[1], s1[2] * s1[3]), lambda i: (i, 0, 0)),
             pl.BlockSpec((1, s2[1], s2[2] * s2[3]), lambda i: (i, 0, 0))]
            + [w_spec(s[1]) for s in shapes]
            + [pl.BlockSpec((3, NA, NO), lambda i: (0, 0, 0))]
        ),
        out_specs=[
            pl.BlockSpec((1, NA, s0[2], s0[3], NO), lambda i: (i, 0, 0, 0, 0)),
            pl.BlockSpec((1, NA, s1[2] * s1[3], NO), lambda i: (i, 0, 0, 0)),
            pl.BlockSpec((1, NA, s2[2] * s2[3], NO), lambda i: (i, 0, 0, 0)),
        ],
        out_shape=[
            jax.ShapeDtypeStruct((bs, NA, s0[2], s0[3], NO), jnp.float32),
            jax.ShapeDtypeStruct((bs, NA, s1[2] * s1[3], NO), jnp.float32),
            jax.ShapeDtypeStruct((bs, NA, s2[2] * s2[3], NO), jnp.float32),
        ],
    )(x0, xr[0], xr[1], *wts, brs)
    y0 = outs[0]
    y1 = outs[1].reshape(bs, NA, s1[2], s1[3], NO)
    y2 = outs[2].reshape(bs, NA, s2[2], s2[3], NO)
    return y0, y1, y2


def kernel(x0, x1, x2, W0, b0, W1, b1, W2, b2):
    return _detect(x0, x1, x2, W0, b0, W1, b1, W2, b2)


# speedup vs baseline: 2.4569x; 1.3945x over previous
import jax
import jax.numpy as jnp
from jax.experimental import pallas as pl

NA = 3
NO = 85


def _detect_kernel(x0_ref, x1_ref, x2_ref,
                   wt0_ref, wt1_ref, wt2_ref, b_ref,
                   out0_ref, out1_ref, out2_ref):
    dn = (((0,), (0,)), ((), ()))
    for x_ref, wt_ref, lvl, out_ref in (
            (x0_ref, wt0_ref, 0, out0_ref),
            (x1_ref, wt1_ref, 1, out1_ref),
            (x2_ref, wt2_ref, 2, out2_ref)):
        xv = x_ref[0].astype(jnp.bfloat16)
        for a in range(NA):
            z = jax.lax.dot_general(xv, wt_ref[a], dn,
                                    preferred_element_type=jnp.float32)
            out_ref[0, a] = z + b_ref[lvl, a]


def _pack_w(W, b):
    c = W.shape[1]
    wt = W.reshape(NA, NO, c).transpose(0, 2, 1).astype(jnp.bfloat16)
    br = b.reshape(NA, NO)
    return wt, br


@jax.jit
def _detect(x0, x1, x2, W0, b0, W1, b1, W2, b2):
    bs = x0.shape[0]
    shapes = [x.shape for x in (x0, x1, x2)]
    xr = [x.reshape(x.shape[0], x.shape[1], -1) for x in (x0, x1, x2)]
    packed = [_pack_w(W, b) for W, b in ((W0, b0), (W1, b1), (W2, b2))]
    wts = [p[0] for p in packed]
    brs = jnp.stack([p[1] for p in packed])

    def x_spec(c, hw):
        return pl.BlockSpec((1, c, hw), lambda i: (i, 0, 0))

    def w_spec(c):
        return pl.BlockSpec((NA, c, NO), lambda i: (0, 0, 0))

    def o_spec(hw):
        return pl.BlockSpec((1, NA, hw, NO), lambda i: (i, 0, 0, 0))

    outs = pl.pallas_call(
        _detect_kernel,
        grid=(bs,),
        in_specs=(
            [x_spec(s[1], s[2] * s[3]) for s in shapes]
            + [w_spec(s[1]) for s in shapes]
            + [pl.BlockSpec((3, NA, NO), lambda i: (0, 0, 0))]
        ),
        out_specs=[o_spec(s[2] * s[3]) for s in shapes],
        out_shape=[
            jax.ShapeDtypeStruct((bs, NA, s[2] * s[3], NO), jnp.float32)
            for s in shapes],
    )(*xr, *wts, brs)
    return tuple(
        o.reshape(bs, NA, s[2], s[3], NO) for o, s in zip(outs, shapes))


def kernel(x0, x1, x2, W0, b0, W1, b1, W2, b2):
    return _detect(x0, x1, x2, W0, b0, W1, b1, W2, b2)
